# baseline (speedup 1.0000x reference)
import jax
import jax.numpy as jnp
from jax import lax
from jax.experimental import pallas as pl
from jax.experimental.pallas import tpu as pltpu
from jax.experimental.pallas import tpu_sc as plsc

B = 16384
L = 20
LP = 64
LS = 24
D = 64
NC = 2
NS = 16
NW = NC * NS
RW = B // NW
C = 32
NCHUNK = RW // C


def _body(tok_tab, typ_tab, ids, typ_idx, scales, out,
          ids_v, typ_idx_v, tok_rows_v, typ_rows_v, out_v, scale_v,
          is0, is1, ts0, ts1, ys0, ys1, os0, os1):
    ids_sems = (is0, is1)
    tok_sems = (ts0, ts1)
    typ_sems = (ys0, ys1)
    out_sems = (os0, os1)
    wid = lax.axis_index("s") * NC + lax.axis_index("c")
    pltpu.sync_copy(scales, scale_v)
    s_typ = scale_v[0, :]
    s_tok = scale_v[1, :]
    pltpu.sync_copy(typ_idx.at[pl.ds(wid * RW, RW)], typ_idx_v)

    def issue_ids(k, b):
        base = wid * RW + k * C
        pltpu.async_copy(ids.at[pl.ds(base, C)], ids_v.at[b], ids_sems[b])

    def wait_ids(b):
        pltpu.make_async_copy(ids.at[pl.ds(0, C)], ids_v.at[b],
                              ids_sems[b]).wait()

    def issue_gathers(k, b):
        for r in range(C):
            pltpu.async_copy(tok_tab.at[ids_v.at[b, r, pl.ds(0, LS)]],
                             tok_rows_v.at[b, pl.ds(r * LS, LS)],
                             tok_sems[b])
        pltpu.async_copy(typ_tab.at[typ_idx_v.at[pl.ds(k * C, C)]],
                         typ_rows_v.at[b], typ_sems[b])

    def wait_gathers(b):
        pltpu.make_async_copy(tok_tab.at[pl.ds(0, C * LS)],
                              tok_rows_v.at[b], tok_sems[b]).wait()
        pltpu.make_async_copy(typ_tab.at[pl.ds(0, C)],
                              typ_rows_v.at[b], typ_sems[b]).wait()

    def wait_out(b):
        pltpu.make_async_copy(out_v.at[b], out.at[pl.ds(0, C)],
                              out_sems[b]).wait()

    def compute(b):
        @plsc.parallel_loop(0, C, step=1, unroll=2)
        def row(r):
            rb = r * LS
            for c in range(D // 16):
                sl = pl.ds(c * 16, 16)
                vs = [tok_rows_v[b, rb + j, sl] for j in range(L)]
                while len(vs) > 1:
                    nxt = [vs[i] + vs[i + 1] for i in range(0, len(vs) - 1, 2)]
                    if len(vs) % 2:
                        nxt.append(vs[-1])
                    vs = nxt
                out_v[b, r, sl] = typ_rows_v[b, r, sl] * s_typ
                out_v[b, r, pl.ds(D + c * 16, 16)] = vs[0] * s_tok

    issue_ids(0, 0)
    issue_ids(1, 1)
    wait_ids(0)
    issue_gathers(0, 0)

    def pair(k2, carry):
        for b in range(2):
            k = k2 * 2 + b

            @pl.when(k + 1 < NCHUNK)
            def _():
                wait_ids(1 - b)
                issue_gathers(k + 1, 1 - b)

            wait_gathers(b)

            @pl.when(k + 2 < NCHUNK)
            def _():
                issue_ids(k + 2, b)

            @pl.when(k >= 2)
            def _():
                wait_out(b)

            compute(b)
            base = wid * RW + k * C
            pltpu.async_copy(out_v.at[b], out.at[pl.ds(base, C)], out_sems[b])
        return carry

    lax.fori_loop(0, NCHUNK // 2, pair, 0)
    wait_out(0)
    wait_out(1)


def kernel(type_index, sub_token_ids, reduce_dim, concat_dim, token_table, type_table):
    ids64 = jnp.pad(sub_token_ids, ((0, 0), (0, LP - L)))
    s_typ = jnp.float32(concat_dim)
    s_tok = jnp.float32(reduce_dim) * jnp.float32(concat_dim)
    scales = jnp.stack([jnp.full((16,), s_typ, jnp.float32),
                        jnp.full((16,), s_tok, jnp.float32)])
    mesh = plsc.VectorSubcoreMesh(core_axis_name="c", subcore_axis_name="s",
                                  num_cores=NC, num_subcores=NS)
    f = pl.kernel(
        _body,
        out_type=jax.ShapeDtypeStruct((B, 2 * D), jnp.float32),
        mesh=mesh,
        compiler_params=pltpu.CompilerParams(use_tc_tiling_on_sc=False),
        scratch_types=[
            pltpu.VMEM((2, C, LP), jnp.int32),
            pltpu.VMEM((RW,), jnp.int32),
            pltpu.VMEM((2, C * LS, D), jnp.float32),
            pltpu.VMEM((2, C, D), jnp.float32),
            pltpu.VMEM((2, C, 2 * D), jnp.float32),
            pltpu.VMEM((2, 16), jnp.float32),
            pltpu.SemaphoreType.DMA,
            pltpu.SemaphoreType.DMA,
            pltpu.SemaphoreType.DMA,
            pltpu.SemaphoreType.DMA,
            pltpu.SemaphoreType.DMA,
            pltpu.SemaphoreType.DMA,
            pltpu.SemaphoreType.DMA,
            pltpu.SemaphoreType.DMA,
        ],
    )
    return f(token_table, type_table, ids64, type_index, scales)


# speedup vs baseline: 11.5542x; 11.5542x over previous
import jax
import jax.numpy as jnp
from jax import lax
from jax.experimental import pallas as pl
from jax.experimental.pallas import tpu as pltpu
from jax.experimental.pallas import tpu_sc as plsc

B = 16384
L = 20
D = 64
NC = 2
NS = 16
NW = NC * NS
RW = B // NW
C = 32
NCHUNK = RW // C


def _body(tok_tab, typ_tab, ids_t, typ_idx, scales, out,
          tok_idx_v, typ_idx_v, tok_rows_v, typ_rows_v, out_v, scale_v,
          ts0, ts1, ys0, ys1, os0, os1):
    tok_sems = (ts0, ts1)
    typ_sems = (ys0, ys1)
    out_sems = (os0, os1)
    wid = lax.axis_index("s") * NC + lax.axis_index("c")
    pltpu.sync_copy(scales, scale_v)
    s_typ = scale_v[0, :]
    s_tok = scale_v[1, :]
    pltpu.sync_copy(ids_t.at[:, pl.ds(wid * RW, RW)], tok_idx_v)
    pltpu.sync_copy(typ_idx.at[pl.ds(wid * RW, RW)], typ_idx_v)

    def issue(k, b):
        for j in range(L):
            pltpu.async_copy(tok_tab.at[tok_idx_v.at[j, pl.ds(k * C, C)]],
                             tok_rows_v.at[b, pl.ds(j * C, C)],
                             tok_sems[b])
        pltpu.async_copy(typ_tab.at[typ_idx_v.at[pl.ds(k * C, C)]],
                         typ_rows_v.at[b], typ_sems[b])

    def wait_gathers(b):
        pltpu.make_async_copy(tok_tab.at[pl.ds(0, C * L)],
                              tok_rows_v.at[b], tok_sems[b]).wait()
        pltpu.make_async_copy(typ_tab.at[pl.ds(0, C)],
                              typ_rows_v.at[b], typ_sems[b]).wait()

    def wait_out(b):
        pltpu.make_async_copy(out_v.at[b], out.at[pl.ds(0, C)],
                              out_sems[b]).wait()

    def compute(b):
        @plsc.parallel_loop(0, C, step=1, unroll=4)
        def row(r):
            for c in range(D // 16):
                sl = pl.ds(c * 16, 16)
                vs = [tok_rows_v[b, j * C + r, sl] for j in range(L)]
                while len(vs) > 1:
                    nxt = [vs[i] + vs[i + 1] for i in range(0, len(vs) - 1, 2)]
                    if len(vs) % 2:
                        nxt.append(vs[-1])
                    vs = nxt
                out_v[b, r, sl] = typ_rows_v[b, r, sl] * s_typ
                out_v[b, r, pl.ds(D + c * 16, 16)] = vs[0] * s_tok

    issue(0, 0)

    def pair(k2, carry):
        for b in range(2):
            k = k2 * 2 + b

            @pl.when(k + 1 < NCHUNK)
            def _():
                issue(k + 1, 1 - b)

            wait_gathers(b)

            @pl.when(k >= 2)
            def _():
                wait_out(b)

            compute(b)
            base = wid * RW + k * C
            pltpu.async_copy(out_v.at[b], out.at[pl.ds(base, C)], out_sems[b])
        return carry

    lax.fori_loop(0, NCHUNK // 2, pair, 0)
    wait_out(0)
    wait_out(1)


def kernel(type_index, sub_token_ids, reduce_dim, concat_dim, token_table, type_table):
    s_typ = jnp.float32(concat_dim)
    s_tok = jnp.float32(reduce_dim) * jnp.float32(concat_dim)
    scales = jnp.stack([jnp.full((16,), s_typ, jnp.float32),
                        jnp.full((16,), s_tok, jnp.float32)])
    mesh = plsc.VectorSubcoreMesh(core_axis_name="c", subcore_axis_name="s",
                                  num_cores=NC, num_subcores=NS)
    f = pl.kernel(
        _body,
        out_type=jax.ShapeDtypeStruct((B, 2 * D), jnp.float32),
        mesh=mesh,
        compiler_params=pltpu.CompilerParams(use_tc_tiling_on_sc=False),
        scratch_types=[
            pltpu.VMEM((L, RW), jnp.int32),
            pltpu.VMEM((RW,), jnp.int32),
            pltpu.VMEM((2, C * L, D), jnp.float32),
            pltpu.VMEM((2, C, D), jnp.float32),
            pltpu.VMEM((2, C, 2 * D), jnp.float32),
            pltpu.VMEM((2, 16), jnp.float32),
            pltpu.SemaphoreType.DMA,
            pltpu.SemaphoreType.DMA,
            pltpu.SemaphoreType.DMA,
            pltpu.SemaphoreType.DMA,
            pltpu.SemaphoreType.DMA,
            pltpu.SemaphoreType.DMA,
        ],
    )
    return f(token_table, type_table, sub_token_ids.T, type_index, scales)
